# baseline (speedup 1.0000x reference)
import jax
import jax.numpy as jnp
from jax.experimental import pallas as pl
from jax.experimental.pallas import tpu as pltpu


def _round_up(x, m):
    return ((x + m - 1) // m) * m


def _ffn_kernel(x_ref, w1_ref, b1_ref, w2_ref, b2_ref, o_ref):
    h = jnp.dot(x_ref[...], w1_ref[...], preferred_element_type=jnp.float32)
    h = jnp.maximum(h + b1_ref[...], 0.0)
    o = jnp.dot(h.astype(jnp.bfloat16), w2_ref[...],
                preferred_element_type=jnp.float32)
    o_ref[...] = (o + b2_ref[...]).astype(o_ref.dtype)


@jax.jit
def kernel(x, w1, b1, w2, b2):
    B, S, D = x.shape
    F = w1.shape[1]
    N = B * S

    x2d = x.reshape(N, D).astype(jnp.bfloat16)
    w1b = w1.astype(jnp.bfloat16)
    w2b = w2.astype(jnp.bfloat16)
    b1_2d = b1.reshape(1, F)
    b2_2d = b2.reshape(1, D)

    tm = min(256, _round_up(N, 8))
    padded_N = _round_up(N, tm)
    if padded_N != N:
        x2d = jnp.pad(x2d, ((0, padded_N - N), (0, 0)))
    grid = (padded_N // tm,)

    cost = pl.CostEstimate(
        flops=4 * padded_N * D * F,
        transcendentals=0,
        bytes_accessed=(padded_N * D * 2
                        + padded_N * D * 4
                        + 2 * D * F * 2
                        + (F + D) * 4),
    )

    out2d = pl.pallas_call(
        _ffn_kernel,
        out_shape=jax.ShapeDtypeStruct((padded_N, D), x.dtype),
        grid=grid,
        in_specs=[
            pl.BlockSpec((tm, D), lambda i: (i, 0)),
            pl.BlockSpec((D, F), lambda i: (0, 0)),
            pl.BlockSpec((1, F), lambda i: (0, 0)),
            pl.BlockSpec((F, D), lambda i: (0, 0)),
            pl.BlockSpec((1, D), lambda i: (0, 0)),
        ],
        out_specs=pl.BlockSpec((tm, D), lambda i: (i, 0)),
        compiler_params=pltpu.CompilerParams(
            dimension_semantics=("parallel",),
            vmem_limit_bytes=60 * 1024 * 1024,
        ),
        cost_estimate=cost,
    )(x2d, w1b, b1_2d, w2b, b2_2d)

    return out2d[:N].reshape(B, S, D)


# speedup vs baseline: 1.2290x; 1.2290x over previous
import jax
import jax.numpy as jnp
from jax.experimental import pallas as pl
from jax.experimental.pallas import tpu as pltpu


def _round_up(x, m):
    return ((x + m - 1) // m) * m


def _ffn_kernel(x_ref, w1_ref, b1_ref, w2_ref, b2_ref, o_ref):
    h = jnp.dot(x_ref[...], w1_ref[...], preferred_element_type=jnp.float32)
    h = jnp.maximum(h + b1_ref[...], 0.0)
    o = jnp.dot(h, w2_ref[...], preferred_element_type=jnp.float32)
    o_ref[...] = o + b2_ref[...]


@jax.jit
def kernel(x, w1, b1, w2, b2):
    B, S, D = x.shape
    F = w1.shape[1]
    N = B * S

    x2d = x.reshape(N, D)
    b1_2d = b1.reshape(1, F)
    b2_2d = b2.reshape(1, D)

    tm = min(256, _round_up(N, 8))
    padded_N = _round_up(N, tm)
    if padded_N != N:
        x2d = jnp.pad(x2d, ((0, padded_N - N), (0, 0)))
    grid = (padded_N // tm,)

    cost = pl.CostEstimate(
        flops=4 * padded_N * D * F,
        transcendentals=0,
        bytes_accessed=(padded_N * D * 4
                        + padded_N * D * 4
                        + 2 * D * F * 4
                        + (F + D) * 4),
    )

    out2d = pl.pallas_call(
        _ffn_kernel,
        out_shape=jax.ShapeDtypeStruct((padded_N, D), x.dtype),
        grid=grid,
        in_specs=[
            pl.BlockSpec((tm, D), lambda i: (i, 0)),
            pl.BlockSpec((D, F), lambda i: (0, 0)),
            pl.BlockSpec((1, F), lambda i: (0, 0)),
            pl.BlockSpec((F, D), lambda i: (0, 0)),
            pl.BlockSpec((1, D), lambda i: (0, 0)),
        ],
        out_specs=pl.BlockSpec((tm, D), lambda i: (i, 0)),
        compiler_params=pltpu.CompilerParams(
            dimension_semantics=("parallel",),
            vmem_limit_bytes=60 * 1024 * 1024,
        ),
        cost_estimate=cost,
    )(x2d, w1, b1_2d, w2, b2_2d)

    return out2d[:N].reshape(B, S, D)


# speedup vs baseline: 1.3564x; 1.1037x over previous
import jax
import jax.numpy as jnp
from jax.experimental import pallas as pl
from jax.experimental.pallas import tpu as pltpu


def _round_up(x, m):
    return ((x + m - 1) // m) * m


def _make_ffn_kernel(nc, tf):
    def _ffn_kernel(x_ref, w1_hbm, b1_ref, w2_hbm, b2_ref, o_ref,
                    w1_vmem, w2_vmem, sem1, sem2):
        i = pl.program_id(0)

        def _w1_copy(c):
            return pltpu.make_async_copy(
                w1_hbm.at[:, pl.ds(c * tf, tf)],
                w1_vmem.at[:, pl.ds(c * tf, tf)],
                sem1.at[c])

        def _w2_copy(c):
            return pltpu.make_async_copy(
                w2_hbm.at[pl.ds(c * tf, tf), :],
                w2_vmem.at[pl.ds(c * tf, tf), :],
                sem2.at[c])

        @pl.when(i == 0)
        def _():
            for c in range(nc):
                _w1_copy(c).start()
                _w2_copy(c).start()
            x_blk = x_ref[...]
            for c in range(nc):
                _w1_copy(c).wait()
                _w2_copy(c).wait()
                h = jnp.dot(x_blk, w1_vmem[:, c * tf:(c + 1) * tf],
                            preferred_element_type=jnp.float32)
                h = jnp.maximum(h + b1_ref[:, c * tf:(c + 1) * tf], 0.0)
                part = jnp.dot(h, w2_vmem[c * tf:(c + 1) * tf, :],
                               preferred_element_type=jnp.float32)
                if c == 0:
                    o_ref[...] = part + b2_ref[...]
                else:
                    o_ref[...] = o_ref[...] + part

        @pl.when(i > 0)
        def _():
            h = jnp.dot(x_ref[...], w1_vmem[...],
                        preferred_element_type=jnp.float32)
            h = jnp.maximum(h + b1_ref[...], 0.0)
            o = jnp.dot(h, w2_vmem[...], preferred_element_type=jnp.float32)
            o_ref[...] = o + b2_ref[...]

    return _ffn_kernel


@jax.jit
def kernel(x, w1, b1, w2, b2):
    B, S, D = x.shape
    F = w1.shape[1]
    N = B * S

    x2d = x.reshape(N, D)
    b1_2d = b1.reshape(1, F)
    b2_2d = b2.reshape(1, D)

    tm = min(512, _round_up(N, 8))
    padded_N = _round_up(N, tm)
    if padded_N != N:
        x2d = jnp.pad(x2d, ((0, padded_N - N), (0, 0)))
    ni = padded_N // tm

    tf = 512
    while F % tf != 0:
        tf //= 2
    nc = F // tf

    cost = pl.CostEstimate(
        flops=4 * padded_N * D * F,
        transcendentals=0,
        bytes_accessed=(padded_N * D * 4
                        + padded_N * D * 4
                        + 2 * D * F * 4
                        + (F + D) * 4),
    )

    out2d = pl.pallas_call(
        _make_ffn_kernel(nc, tf),
        out_shape=jax.ShapeDtypeStruct((padded_N, D), x.dtype),
        grid=(ni,),
        in_specs=[
            pl.BlockSpec((tm, D), lambda i: (i, 0)),
            pl.BlockSpec(memory_space=pl.ANY),
            pl.BlockSpec((1, F), lambda i: (0, 0)),
            pl.BlockSpec(memory_space=pl.ANY),
            pl.BlockSpec((1, D), lambda i: (0, 0)),
        ],
        out_specs=pl.BlockSpec((tm, D), lambda i: (i, 0)),
        scratch_shapes=[
            pltpu.VMEM((D, F), jnp.float32),
            pltpu.VMEM((F, D), jnp.float32),
            pltpu.SemaphoreType.DMA((nc,)),
            pltpu.SemaphoreType.DMA((nc,)),
        ],
        compiler_params=pltpu.CompilerParams(
            dimension_semantics=("arbitrary",),
            vmem_limit_bytes=60 * 1024 * 1024,
        ),
        cost_estimate=cost,
    )(x2d, w1, b1_2d, w2, b2_2d)

    return out2d[:N].reshape(B, S, D)
